# Initial kernel scaffold; baseline (speedup 1.0000x reference)
import functools

import jax
import jax.numpy as jnp
from jax import lax
from jax.experimental import pallas as pl
from jax.experimental.pallas import tpu as pltpu
from jax.experimental.pallas import tpu_sc as plsc

NUM_CORES = 2
NUM_SUBCORES = 16
NUM_WORKERS = NUM_CORES * NUM_SUBCORES


def _make_sc_gather(B, V, D, b_per_w, chunk):
  nchunks = b_per_w // chunk
  mesh = plsc.VectorSubcoreMesh(
      core_axis_name="c", subcore_axis_name="s",
      num_cores=NUM_CORES, num_subcores=NUM_SUBCORES)

  @functools.partial(
      pl.kernel,
      out_type=jax.ShapeDtypeStruct((B, D), jnp.float32),
      mesh=mesh,
      scratch_types=[
          pltpu.VMEM((b_per_w,), jnp.int32),
          pltpu.VMEM((chunk, D), jnp.float32),
          pltpu.SemaphoreType.DMA,
      ],
  )
  def sc_gather(ids_hbm, table_hbm, out_hbm, idx_v, rows_v, sem):
    wid = lax.axis_index("s") * NUM_CORES + lax.axis_index("c")
    base = pl.multiple_of(wid * b_per_w, b_per_w)
    pltpu.sync_copy(ids_hbm.at[pl.ds(base, b_per_w)], idx_v)

    def body(c, _):
      off = pl.multiple_of(c * chunk, chunk)
      idx = idx_v.at[pl.ds(off, chunk)]
      pltpu.async_copy(table_hbm.at[idx], rows_v, sem).wait()
      pltpu.sync_copy(rows_v, out_hbm.at[pl.ds(base + off, chunk)])
      return 0

    lax.fori_loop(0, nchunks, body, 0)

  return sc_gather


def kernel(word_ids, table):
  B = word_ids.shape[0]
  V, D = table.shape
  ids = word_ids[:, 0].astype(jnp.int32)
  f = _make_sc_gather(B, V, D, B // NUM_WORKERS, 128)
  return f(ids, table)


# speedup vs baseline: 54.0392x; 54.0392x over previous
import functools

import jax
import jax.numpy as jnp
from jax import lax
from jax.experimental import pallas as pl
from jax.experimental.pallas import tpu as pltpu
from jax.experimental.pallas import tpu_sc as plsc

NUM_CORES = 2
NUM_SUBCORES = 16
NUM_WORKERS = NUM_CORES * NUM_SUBCORES


def _make_sc_gather(B, V, D, b_per_w, chunk):
  nchunks = b_per_w // chunk
  mesh = plsc.VectorSubcoreMesh(
      core_axis_name="c", subcore_axis_name="s",
      num_cores=NUM_CORES, num_subcores=NUM_SUBCORES)

  @functools.partial(
      pl.kernel,
      out_type=jax.ShapeDtypeStruct((B, D), jnp.float32),
      mesh=mesh,
      scratch_types=[
          pltpu.VMEM((b_per_w,), jnp.int32),
          pltpu.VMEM((chunk, D), jnp.float32),
          pltpu.SemaphoreType.DMA,
      ],
      compiler_params=pltpu.CompilerParams(use_tc_tiling_on_sc=False),
  )
  def sc_gather(ids_hbm, table_hbm, out_hbm, idx_v, rows_v, sem):
    wid = lax.axis_index("s") * NUM_CORES + lax.axis_index("c")
    base = pl.multiple_of(wid * b_per_w, b_per_w)
    pltpu.sync_copy(ids_hbm.at[pl.ds(base, b_per_w)], idx_v)

    def body(c, _):
      off = pl.multiple_of(c * chunk, chunk)
      idx = idx_v.at[pl.ds(off, chunk)]
      pltpu.async_copy(table_hbm.at[idx], rows_v, sem).wait()
      pltpu.sync_copy(rows_v, out_hbm.at[pl.ds(base + off, chunk)])
      return 0

    lax.fori_loop(0, nchunks, body, 0)

  return sc_gather


def kernel(word_ids, table):
  B = word_ids.shape[0]
  V, D = table.shape
  ids = word_ids[:, 0].astype(jnp.int32)
  f = _make_sc_gather(B, V, D, B // NUM_WORKERS, 128)
  return f(ids, table)
